# baseline (speedup 1.0000x reference)
import jax
import jax.numpy as jnp
from jax.experimental import pallas as pl
from jax.experimental.pallas import tpu as pltpu

NUM_EXPERTS = 64
TOP_K = 2
HIDDEN = 1024
FFN = 512
NW = 4


def _routing_body(x_ref, gate_ref, w_ref, ids_ref, flags_ref):
    x = x_ref[...]
    gate = gate_ref[...]
    logits = jax.lax.dot_general(
        x, gate, (((1,), (1,)), ((), ())), preferred_element_type=jnp.float32
    )
    T, E = logits.shape
    e_iota = jax.lax.broadcasted_iota(jnp.int32, (T, E), 1)

    l1 = jnp.max(logits, axis=-1, keepdims=True)
    i1 = jnp.min(jnp.where(logits == l1, e_iota, E), axis=-1, keepdims=True)
    masked = jnp.where(e_iota == i1, -jnp.inf, logits)
    l2 = jnp.max(masked, axis=-1, keepdims=True)
    i2 = jnp.min(jnp.where(masked == l2, e_iota, E), axis=-1, keepdims=True)
    w1 = 1.0 / (1.0 + jnp.exp(l2 - l1))
    w2 = 1.0 - w1
    w_dense = (jnp.where(e_iota == i1, w1, 0.0)
               + jnp.where(e_iota == i2, w2, 0.0))
    w_ref[...] = w_dense
    active_row = jnp.sum((w_dense > 0.0).astype(jnp.int32),
                         axis=0, keepdims=True) > 0

    logits_t = jax.lax.dot_general(
        gate, x, (((1,), (1,)), ((), ())), preferred_element_type=jnp.float32
    )
    et_iota = jax.lax.broadcasted_iota(jnp.int32, (E, T), 0)
    l1c = jnp.max(logits_t, axis=0, keepdims=True)
    i1c = jnp.min(jnp.where(logits_t == l1c, et_iota, E), axis=0, keepdims=True)
    masked_c = jnp.where(et_iota == i1c, -jnp.inf, logits_t)
    l2c = jnp.max(masked_c, axis=0, keepdims=True)
    i2c = jnp.min(jnp.where(masked_c == l2c, et_iota, E), axis=0, keepdims=True)
    routed_t = (et_iota == i1c) | (et_iota == i2c)
    active_col = jnp.sum(routed_t.astype(jnp.int32),
                         axis=1, keepdims=True) > 0

    e_row = jax.lax.broadcasted_iota(jnp.int32, (1, E), 1)
    e_col = jax.lax.broadcasted_iota(jnp.int32, (E, 1), 0)
    key_row = jnp.where(active_row, e_row, e_row + E)
    key_col = jnp.where(active_col, e_col, e_col + E)
    rank_col = jnp.sum((key_col > key_row).astype(jnp.int32),
                       axis=1, keepdims=True)
    hit = (rank_col == e_row).astype(jnp.int32)
    perm = jnp.sum(hit * e_col, axis=0, keepdims=True)
    flags = jnp.sum(hit * active_col.astype(jnp.int32),
                    axis=0, keepdims=True)
    n_act = jnp.sum(active_col.astype(jnp.int32), axis=0, keepdims=True)
    lastk = []
    prev = jnp.max(jnp.where(active_row, e_row, 0), axis=1, keepdims=True)
    lastk.append(prev)
    for k in range(1, NW):
        cand = jnp.max(jnp.where(active_row & (e_row < prev), e_row, 0),
                       axis=1, keepdims=True)
        cand = jnp.where(n_act > k, cand, prev)
        lastk.append(cand)
        prev = cand
    back = (n_act - 1 - e_row) & (NW - 1)
    pad_ids = lastk[0]
    for k in range(1, NW):
        pad_ids = jnp.where(back == k, lastk[k], pad_ids)
    ids_ref[...] = jnp.where(flags > 0, perm, pad_ids)
    flags_ref[...] = flags


def _expert_contrib(x, w_ref, parts, eid):
    o = None
    for wg, wu, wd in parts:
        g = jax.lax.dot_general(
            x, wg, (((1,), (1,)), ((), ())), preferred_element_type=jnp.float32)
        u = jax.lax.dot_general(
            x, wu, (((1,), (1,)), ((), ())), preferred_element_type=jnp.float32)
        h = (g * jax.nn.sigmoid(g)) * u
        oh = jax.lax.dot_general(
            h, wd, (((1,), (1,)), ((), ())), preferred_element_type=jnp.float32)
        o = oh if o is None else o + oh
    T, E = w_ref.shape
    e_iota = jax.lax.broadcasted_iota(jnp.int32, (T, E), 1)
    w_col = jnp.sum(jnp.where(e_iota == eid, w_ref[...], 0.0),
                    axis=-1, keepdims=True)
    return o * w_col


def _moe_body(ids_ref, flags_ref, x_ref, w_ref, *refs):
    wrefs, out_ref = refs[:-1], refs[-1]
    j = pl.program_id(0)

    @pl.when(j == 0)
    def _init():
        out_ref[...] = jnp.zeros_like(out_ref)

    for k in range(NW):
        r = wrefs[6 * k:6 * k + 6]
        parts = [(r[0][0], r[2][0], r[4][0]), (r[1][0], r[3][0], r[5][0])]

        @pl.when(flags_ref[NW * j + k] > 0)
        def _slot(parts=parts, k=k):
            out_ref[...] += _expert_contrib(
                x_ref[...], w_ref, parts, ids_ref[NW * j + k])


def kernel(hidden_states, gate_w, Wg, Wu, Wd):
    B, S, D = hidden_states.shape
    T = B * S
    E = NUM_EXPERTS
    F = FFN
    x = hidden_states.reshape(T, D)

    w_dense, ids, flags = pl.pallas_call(
        _routing_body,
        out_shape=[
            jax.ShapeDtypeStruct((T, E), jnp.float32),
            jax.ShapeDtypeStruct((1, E), jnp.int32),
            jax.ShapeDtypeStruct((1, E), jnp.int32),
        ],
    )(x, gate_w)
    ids = ids.reshape(E)
    flags = flags.reshape(E)

    def _wspec(k, shape, idx):
        return pl.BlockSpec(
            (1,) + shape, lambda j, ids, flags, k=k, idx=idx:
                (ids[NW * j + k],) + idx)

    weight_specs = []
    weight_args = []
    for k in range(NW):
        weight_specs += [_wspec(k, (F // 2, D), (0, 0)),
                         _wspec(k, (F // 2, D), (1, 0)),
                         _wspec(k, (F // 2, D), (0, 0)),
                         _wspec(k, (F // 2, D), (1, 0)),
                         _wspec(k, (D, F // 2), (0, 0)),
                         _wspec(k, (D, F // 2), (0, 1))]
        weight_args += [Wg, Wg, Wu, Wu, Wd, Wd]

    out = pl.pallas_call(
        _moe_body,
        grid_spec=pltpu.PrefetchScalarGridSpec(
            num_scalar_prefetch=2,
            grid=(E // NW,),
            in_specs=[
                pl.BlockSpec((T, D), lambda j, ids, flags: (0, 0)),
                pl.BlockSpec((T, E), lambda j, ids, flags: (0, 0)),
            ] + weight_specs,
            out_specs=pl.BlockSpec((T, D), lambda j, ids, flags: (0, 0)),
        ),
        out_shape=jax.ShapeDtypeStruct((T, D), jnp.float32),
    )(ids, flags, x, w_dense, *weight_args)

    return out.reshape(B, S, D)


# speedup vs baseline: 1.2172x; 1.2172x over previous
import jax
import jax.numpy as jnp
from jax.experimental import pallas as pl
from jax.experimental.pallas import tpu as pltpu

NUM_EXPERTS = 64
TOP_K = 2
HIDDEN = 1024
FFN = 512
NW = 4


def _routing_body(x_ref, gate_ref, w_ref, ids_ref, flags_ref):
    x = x_ref[...]
    gate = gate_ref[...]
    logits = jax.lax.dot_general(
        x, gate, (((1,), (1,)), ((), ())), preferred_element_type=jnp.float32
    )
    T, E = logits.shape
    e_iota = jax.lax.broadcasted_iota(jnp.int32, (T, E), 1)

    l1 = jnp.max(logits, axis=-1, keepdims=True)
    i1 = jnp.min(jnp.where(logits == l1, e_iota, E), axis=-1, keepdims=True)
    masked = jnp.where(e_iota == i1, -jnp.inf, logits)
    l2 = jnp.max(masked, axis=-1, keepdims=True)
    i2 = jnp.min(jnp.where(masked == l2, e_iota, E), axis=-1, keepdims=True)
    w1 = 1.0 / (1.0 + jnp.exp(l2 - l1))
    w2 = 1.0 - w1
    w_dense = (jnp.where(e_iota == i1, w1, 0.0)
               + jnp.where(e_iota == i2, w2, 0.0))
    w_ref[...] = w_dense
    active_row = jnp.sum((w_dense > 0.0).astype(jnp.int32),
                         axis=0, keepdims=True) > 0

    logits_t = jax.lax.dot_general(
        gate, x, (((1,), (1,)), ((), ())), preferred_element_type=jnp.float32
    )
    et_iota = jax.lax.broadcasted_iota(jnp.int32, (E, T), 0)
    l1c = jnp.max(logits_t, axis=0, keepdims=True)
    i1c = jnp.min(jnp.where(logits_t == l1c, et_iota, E), axis=0, keepdims=True)
    masked_c = jnp.where(et_iota == i1c, -jnp.inf, logits_t)
    l2c = jnp.max(masked_c, axis=0, keepdims=True)
    i2c = jnp.min(jnp.where(masked_c == l2c, et_iota, E), axis=0, keepdims=True)
    routed_t = (et_iota == i1c) | (et_iota == i2c)
    active_col = jnp.sum(routed_t.astype(jnp.int32),
                         axis=1, keepdims=True) > 0

    e_row = jax.lax.broadcasted_iota(jnp.int32, (1, E), 1)
    e_col = jax.lax.broadcasted_iota(jnp.int32, (E, 1), 0)
    key_row = jnp.where(active_row, e_row, e_row + E)
    key_col = jnp.where(active_col, e_col, e_col + E)
    rank_col = jnp.sum((key_col > key_row).astype(jnp.int32),
                       axis=1, keepdims=True)
    hit = (rank_col == e_row).astype(jnp.int32)
    perm = jnp.sum(hit * e_col, axis=0, keepdims=True)
    flags = jnp.sum(hit * active_col.astype(jnp.int32),
                    axis=0, keepdims=True)
    n_act = jnp.sum(active_col.astype(jnp.int32), axis=0, keepdims=True)
    lastk = []
    prev = jnp.max(jnp.where(active_row, e_row, 0), axis=1, keepdims=True)
    lastk.append(prev)
    for k in range(1, NW):
        cand = jnp.max(jnp.where(active_row & (e_row < prev), e_row, 0),
                       axis=1, keepdims=True)
        cand = jnp.where(n_act > k, cand, prev)
        lastk.append(cand)
        prev = cand
    back = (n_act - 1 - e_row) & (NW - 1)
    pad_ids = lastk[0]
    for k in range(1, NW):
        pad_ids = jnp.where(back == k, lastk[k], pad_ids)
    ids_ref[...] = jnp.where(flags > 0, perm, pad_ids)
    flags_ref[...] = flags


def _expert_contrib(x, w_ref, wg, wu, wd, eid):
    xb = x.astype(jnp.bfloat16)
    g = jax.lax.dot_general(
        xb, wg.astype(jnp.bfloat16), (((1,), (1,)), ((), ())),
        preferred_element_type=jnp.float32)
    u = jax.lax.dot_general(
        xb, wu.astype(jnp.bfloat16), (((1,), (1,)), ((), ())),
        preferred_element_type=jnp.float32)
    h = (g * jax.nn.sigmoid(g)) * u
    o = jax.lax.dot_general(
        h.astype(jnp.bfloat16), wd.astype(jnp.bfloat16), (((1,), (1,)), ((), ())),
        preferred_element_type=jnp.float32)
    T, E = w_ref.shape
    e_iota = jax.lax.broadcasted_iota(jnp.int32, (T, E), 1)
    w_col = jnp.sum(jnp.where(e_iota == eid, w_ref[...], 0.0),
                    axis=-1, keepdims=True)
    return o * w_col


def _moe_body(ids_ref, flags_ref, x_ref, w_ref, *refs):
    wrefs, out_ref = refs[:-1], refs[-1]
    j = pl.program_id(0)

    @pl.when(j == 0)
    def _init():
        out_ref[...] = jnp.zeros_like(out_ref)

    for k in range(NW):
        wg, wu, wd = wrefs[3 * k:3 * k + 3]

        @pl.when(flags_ref[NW * j + k] > 0)
        def _slot(wg=wg, wu=wu, wd=wd, k=k):
            out_ref[...] += _expert_contrib(
                x_ref[...], w_ref, wg[0], wu[0], wd[0], ids_ref[NW * j + k])


def kernel(hidden_states, gate_w, Wg, Wu, Wd):
    B, S, D = hidden_states.shape
    T = B * S
    E = NUM_EXPERTS
    F = FFN
    x = hidden_states.reshape(T, D)

    w_dense, ids, flags = pl.pallas_call(
        _routing_body,
        out_shape=[
            jax.ShapeDtypeStruct((T, E), jnp.float32),
            jax.ShapeDtypeStruct((1, E), jnp.int32),
            jax.ShapeDtypeStruct((1, E), jnp.int32),
        ],
    )(x, gate_w)
    ids = ids.reshape(E)
    flags = flags.reshape(E)

    def _wspec(k, shape):
        return pl.BlockSpec(
            (1,) + shape, lambda j, ids, flags, k=k: (ids[NW * j + k], 0, 0))

    weight_specs = []
    weight_args = []
    for k in range(NW):
        weight_specs += [_wspec(k, (F, D)), _wspec(k, (F, D)), _wspec(k, (D, F))]
        weight_args += [Wg, Wu, Wd]

    out = pl.pallas_call(
        _moe_body,
        grid_spec=pltpu.PrefetchScalarGridSpec(
            num_scalar_prefetch=2,
            grid=(E // NW,),
            in_specs=[
                pl.BlockSpec((T, D), lambda j, ids, flags: (0, 0)),
                pl.BlockSpec((T, E), lambda j, ids, flags: (0, 0)),
            ] + weight_specs,
            out_specs=pl.BlockSpec((T, D), lambda j, ids, flags: (0, 0)),
        ),
        out_shape=jax.ShapeDtypeStruct((T, D), jnp.float32),
    )(ids, flags, x, w_dense, *weight_args)

    return out.reshape(B, S, D)


# speedup vs baseline: 1.2576x; 1.0332x over previous
import jax
import jax.numpy as jnp
from jax import lax
from jax.experimental import pallas as pl
from jax.experimental.pallas import tpu as pltpu

NUM_EXPERTS = 64
TOP_K = 2
HIDDEN = 1024
FFN = 512
K_BUF = 8


def _routing_body(x_ref, gate_ref, w_ref, ids_ref, n_ref):
    x = x_ref[...]
    gate = gate_ref[...]
    logits = jax.lax.dot_general(
        x, gate, (((1,), (1,)), ((), ())), preferred_element_type=jnp.float32
    )
    T, E = logits.shape
    e_iota = jax.lax.broadcasted_iota(jnp.int32, (T, E), 1)

    l1 = jnp.max(logits, axis=-1, keepdims=True)
    i1 = jnp.min(jnp.where(logits == l1, e_iota, E), axis=-1, keepdims=True)
    masked = jnp.where(e_iota == i1, -jnp.inf, logits)
    l2 = jnp.max(masked, axis=-1, keepdims=True)
    i2 = jnp.min(jnp.where(masked == l2, e_iota, E), axis=-1, keepdims=True)
    w1 = 1.0 / (1.0 + jnp.exp(l2 - l1))
    w2 = 1.0 - w1
    w_dense = (jnp.where(e_iota == i1, w1, 0.0)
               + jnp.where(e_iota == i2, w2, 0.0))
    w_ref[...] = w_dense
    active_row = jnp.sum((w_dense > 0.0).astype(jnp.int32),
                         axis=0, keepdims=True) > 0

    logits_t = jax.lax.dot_general(
        gate, x, (((1,), (1,)), ((), ())), preferred_element_type=jnp.float32
    )
    et_iota = jax.lax.broadcasted_iota(jnp.int32, (E, T), 0)
    l1c = jnp.max(logits_t, axis=0, keepdims=True)
    i1c = jnp.min(jnp.where(logits_t == l1c, et_iota, E), axis=0, keepdims=True)
    masked_c = jnp.where(et_iota == i1c, -jnp.inf, logits_t)
    l2c = jnp.max(masked_c, axis=0, keepdims=True)
    i2c = jnp.min(jnp.where(masked_c == l2c, et_iota, E), axis=0, keepdims=True)
    routed_t = (et_iota == i1c) | (et_iota == i2c)
    active_col = jnp.sum(routed_t.astype(jnp.int32),
                         axis=1, keepdims=True) > 0

    e_row = jax.lax.broadcasted_iota(jnp.int32, (1, E), 1)
    e_col = jax.lax.broadcasted_iota(jnp.int32, (E, 1), 0)
    key_row = jnp.where(active_row, e_row, e_row + E)
    key_col = jnp.where(active_col, e_col, e_col + E)
    rank_col = jnp.sum((key_col > key_row).astype(jnp.int32),
                       axis=1, keepdims=True)
    hit = (rank_col == e_row).astype(jnp.int32)
    perm = jnp.sum(hit * e_col, axis=0, keepdims=True)
    ids_ref[...] = perm
    n_ref[...] = jnp.sum(active_col.astype(jnp.int32), axis=0,
                         keepdims=True)


def _moe_body(ids_ref, n_ref, x_ref, w_ref, wg_hbm, wu_hbm, wd_hbm,
              out_ref, wg_b, wu_b, wd_b, sems):
    n = n_ref[0]

    def start_copies(slot, buf):
        eid = ids_ref[slot]
        pltpu.make_async_copy(wg_hbm.at[eid], wg_b.at[buf], sems.at[buf, 0]).start()
        pltpu.make_async_copy(wu_hbm.at[eid], wu_b.at[buf], sems.at[buf, 1]).start()
        pltpu.make_async_copy(wd_hbm.at[eid], wd_b.at[buf], sems.at[buf, 2]).start()

    for k in range(K_BUF):

        @pl.when(k < n)
        def _prime(k=k):
            start_copies(k, k)

    out_ref[...] = jnp.zeros_like(out_ref)
    x = x_ref[...]
    T, E = w_ref.shape
    e_iota = jax.lax.broadcasted_iota(jnp.int32, (T, E), 1)

    def body(i, carry):
        buf = lax.rem(i, K_BUF)
        pltpu.make_async_copy(wg_hbm.at[0], wg_b.at[buf], sems.at[buf, 0]).wait()
        pltpu.make_async_copy(wu_hbm.at[0], wu_b.at[buf], sems.at[buf, 1]).wait()
        pltpu.make_async_copy(wd_hbm.at[0], wd_b.at[buf], sems.at[buf, 2]).wait()
        g = jax.lax.dot_general(
            x, wg_b[buf], (((1,), (1,)), ((), ())),
            preferred_element_type=jnp.float32)
        u = jax.lax.dot_general(
            x, wu_b[buf], (((1,), (1,)), ((), ())),
            preferred_element_type=jnp.float32)
        h = (g * jax.nn.sigmoid(g)) * u
        o = jax.lax.dot_general(
            h, wd_b[buf], (((1,), (1,)), ((), ())),
            preferred_element_type=jnp.float32)
        w_col = jnp.sum(jnp.where(e_iota == ids_ref[i], w_ref[...], 0.0),
                        axis=-1, keepdims=True)
        out_ref[...] += o * w_col

        @pl.when(i + K_BUF < n)
        def _next():
            start_copies(i + K_BUF, buf)

        return carry

    lax.fori_loop(0, n, body, 0)


def kernel(hidden_states, gate_w, Wg, Wu, Wd):
    B, S, D = hidden_states.shape
    T = B * S
    E = NUM_EXPERTS
    F = FFN
    x = hidden_states.reshape(T, D)

    w_dense, ids, n_act = pl.pallas_call(
        _routing_body,
        out_shape=[
            jax.ShapeDtypeStruct((T, E), jnp.float32),
            jax.ShapeDtypeStruct((1, E), jnp.int32),
            jax.ShapeDtypeStruct((1, 1), jnp.int32),
        ],
    )(x, gate_w)
    ids = ids.reshape(E)
    n_act = n_act.reshape(1)

    out = pl.pallas_call(
        _moe_body,
        grid_spec=pltpu.PrefetchScalarGridSpec(
            num_scalar_prefetch=2,
            grid=(1,),
            in_specs=[
                pl.BlockSpec((T, D), lambda i, ids, n: (0, 0)),
                pl.BlockSpec((T, E), lambda i, ids, n: (0, 0)),
                pl.BlockSpec(memory_space=pl.ANY),
                pl.BlockSpec(memory_space=pl.ANY),
                pl.BlockSpec(memory_space=pl.ANY),
            ],
            out_specs=pl.BlockSpec((T, D), lambda i, ids, n: (0, 0)),
            scratch_shapes=[
                pltpu.VMEM((K_BUF, F, D), jnp.float32),
                pltpu.VMEM((K_BUF, F, D), jnp.float32),
                pltpu.VMEM((K_BUF, D, F), jnp.float32),
                pltpu.SemaphoreType.DMA((K_BUF, 3)),
            ],
        ),
        out_shape=jax.ShapeDtypeStruct((T, D), jnp.float32),
    )(ids, n_act, x, w_dense, Wg, Wu, Wd)

    return out.reshape(B, S, D)


# speedup vs baseline: 1.2859x; 1.0225x over previous
import jax
import jax.numpy as jnp
from jax import lax
from jax.experimental import pallas as pl
from jax.experimental.pallas import tpu as pltpu

NUM_EXPERTS = 64
TOP_K = 2
HIDDEN = 1024
FFN = 512
K_BUF = 8


def _moe_body(x_ref, gate_ref, wg_hbm, wu_hbm, wd_hbm, out_ref,
              ids_vmem, n_vmem, ids_smem, n_smem, w_scr,
              wg_b, wu_b, wd_b, sems, sem_meta):
    x = x_ref[...]
    gate = gate_ref[...]
    logits = jax.lax.dot_general(
        x, gate, (((1,), (1,)), ((), ())), preferred_element_type=jnp.float32
    )
    T, E = logits.shape
    e_iota = jax.lax.broadcasted_iota(jnp.int32, (T, E), 1)

    l1 = jnp.max(logits, axis=-1, keepdims=True)
    i1 = jnp.min(jnp.where(logits == l1, e_iota, E), axis=-1, keepdims=True)
    masked = jnp.where(e_iota == i1, -jnp.inf, logits)
    l2 = jnp.max(masked, axis=-1, keepdims=True)
    i2 = jnp.min(jnp.where(masked == l2, e_iota, E), axis=-1, keepdims=True)
    w1 = 1.0 / (1.0 + jnp.exp(l2 - l1))
    w2 = 1.0 - w1
    w_dense = (jnp.where(e_iota == i1, w1, 0.0)
               + jnp.where(e_iota == i2, w2, 0.0))
    w_scr[...] = w_dense
    active_row = jnp.sum((w_dense > 0.0).astype(jnp.int32),
                         axis=0, keepdims=True) > 0

    logits_t = jax.lax.dot_general(
        gate, x, (((1,), (1,)), ((), ())), preferred_element_type=jnp.float32
    )
    et_iota = jax.lax.broadcasted_iota(jnp.int32, (E, T), 0)
    l1c = jnp.max(logits_t, axis=0, keepdims=True)
    i1c = jnp.min(jnp.where(logits_t == l1c, et_iota, E), axis=0, keepdims=True)
    masked_c = jnp.where(et_iota == i1c, -jnp.inf, logits_t)
    l2c = jnp.max(masked_c, axis=0, keepdims=True)
    i2c = jnp.min(jnp.where(masked_c == l2c, et_iota, E), axis=0, keepdims=True)
    routed_t = (et_iota == i1c) | (et_iota == i2c)
    active_col = jnp.sum(routed_t.astype(jnp.int32),
                         axis=1, keepdims=True) > 0

    e_row = jax.lax.broadcasted_iota(jnp.int32, (1, E), 1)
    e_col = jax.lax.broadcasted_iota(jnp.int32, (E, 1), 0)
    key_row = jnp.where(active_row, e_row, e_row + E)
    key_col = jnp.where(active_col, e_col, e_col + E)
    rank_col = jnp.sum((key_col > key_row).astype(jnp.int32),
                       axis=1, keepdims=True)
    hit = (rank_col == e_row).astype(jnp.int32)
    ids_vmem[...] = jnp.sum(hit * e_col, axis=0, keepdims=True)
    n_vmem[...] = jnp.sum(active_col.astype(jnp.int32), axis=0,
                          keepdims=True)

    pltpu.make_async_copy(ids_vmem, ids_smem, sem_meta).start()
    pltpu.make_async_copy(ids_vmem, ids_smem, sem_meta).wait()
    pltpu.make_async_copy(n_vmem, n_smem, sem_meta).start()
    pltpu.make_async_copy(n_vmem, n_smem, sem_meta).wait()
    n = n_smem[0, 0]

    def start_copies(slot, buf):
        eid = ids_smem[0, slot]
        pltpu.make_async_copy(wg_hbm.at[eid], wg_b.at[buf], sems.at[buf, 0]).start()
        pltpu.make_async_copy(wu_hbm.at[eid], wu_b.at[buf], sems.at[buf, 1]).start()
        pltpu.make_async_copy(wd_hbm.at[eid], wd_b.at[buf], sems.at[buf, 2]).start()

    for k in range(K_BUF):

        @pl.when(k < n)
        def _prime(k=k):
            start_copies(k, k)

    out_ref[...] = jnp.zeros_like(out_ref)

    def body(i, carry):
        buf = lax.rem(i, K_BUF)
        pltpu.make_async_copy(wg_hbm.at[0], wg_b.at[buf], sems.at[buf, 0]).wait()
        pltpu.make_async_copy(wu_hbm.at[0], wu_b.at[buf], sems.at[buf, 1]).wait()
        pltpu.make_async_copy(wd_hbm.at[0], wd_b.at[buf], sems.at[buf, 2]).wait()
        g = jax.lax.dot_general(
            x, wg_b[buf], (((1,), (1,)), ((), ())),
            preferred_element_type=jnp.float32)
        u = jax.lax.dot_general(
            x, wu_b[buf], (((1,), (1,)), ((), ())),
            preferred_element_type=jnp.float32)
        h = (g * jax.nn.sigmoid(g)) * u
        o = jax.lax.dot_general(
            h, wd_b[buf], (((1,), (1,)), ((), ())),
            preferred_element_type=jnp.float32)
        w_col = jnp.sum(jnp.where(e_iota == ids_smem[0, i], w_scr[...], 0.0),
                        axis=-1, keepdims=True)
        out_ref[...] += o * w_col

        @pl.when(i + K_BUF < n)
        def _next():
            start_copies(i + K_BUF, buf)

        return carry

    lax.fori_loop(0, n, body, 0)


def kernel(hidden_states, gate_w, Wg, Wu, Wd):
    B, S, D = hidden_states.shape
    T = B * S
    E = NUM_EXPERTS
    F = FFN
    x = hidden_states.reshape(T, D)

    out = pl.pallas_call(
        _moe_body,
        in_specs=[
            pl.BlockSpec((T, D), lambda: (0, 0)),
            pl.BlockSpec((E, D), lambda: (0, 0)),
            pl.BlockSpec(memory_space=pl.ANY),
            pl.BlockSpec(memory_space=pl.ANY),
            pl.BlockSpec(memory_space=pl.ANY),
        ],
        out_specs=pl.BlockSpec((T, D), lambda: (0, 0)),
        scratch_shapes=[
            pltpu.VMEM((1, E), jnp.int32),
            pltpu.VMEM((1, 1), jnp.int32),
            pltpu.SMEM((1, E), jnp.int32),
            pltpu.SMEM((1, 1), jnp.int32),
            pltpu.VMEM((T, E), jnp.float32),
            pltpu.VMEM((K_BUF, F, D), jnp.float32),
            pltpu.VMEM((K_BUF, F, D), jnp.float32),
            pltpu.VMEM((K_BUF, D, F), jnp.float32),
            pltpu.SemaphoreType.DMA((K_BUF, 3)),
            pltpu.SemaphoreType.DMA,
        ],
        out_shape=jax.ShapeDtypeStruct((T, D), jnp.float32),
    )(x, gate_w, Wg, Wu, Wd)

    return out.reshape(B, S, D)


# speedup vs baseline: 1.3013x; 1.0120x over previous
import jax
import jax.numpy as jnp
from jax import lax
from jax.experimental import pallas as pl
from jax.experimental.pallas import tpu as pltpu

NUM_EXPERTS = 64
TOP_K = 2
HIDDEN = 1024
FFN = 512
K_BUF = 8


def _moe_body(x_ref, gate_ref, wg_hbm, wu_hbm, wd_hbm, out_ref,
              ids_vmem, n_vmem, ids_smem, n_smem, w_scr,
              wg_b, wu_b, wd_b, sems, sem_meta):
    x = x_ref[...]
    gate = gate_ref[...]
    logits = jax.lax.dot_general(
        x, gate, (((1,), (1,)), ((), ())), preferred_element_type=jnp.float32
    )
    T, E = logits.shape
    e_iota = jax.lax.broadcasted_iota(jnp.int32, (T, E), 1)

    l1 = jnp.max(logits, axis=-1, keepdims=True)
    i1 = jnp.min(jnp.where(logits == l1, e_iota, E), axis=-1, keepdims=True)
    masked = jnp.where(e_iota == i1, -jnp.inf, logits)
    l2 = jnp.max(masked, axis=-1, keepdims=True)
    i2 = jnp.min(jnp.where(masked == l2, e_iota, E), axis=-1, keepdims=True)
    w1 = 1.0 / (1.0 + jnp.exp(l2 - l1))
    w2 = 1.0 - w1
    w_dense = (jnp.where(e_iota == i1, w1, 0.0)
               + jnp.where(e_iota == i2, w2, 0.0))
    w_scr[...] = w_dense
    active_row = jnp.sum((w_dense > 0.0).astype(jnp.int32),
                         axis=0, keepdims=True) > 0

    logits_t = jax.lax.dot_general(
        gate, x, (((1,), (1,)), ((), ())), preferred_element_type=jnp.float32
    )
    et_iota = jax.lax.broadcasted_iota(jnp.int32, (E, T), 0)
    l1c = jnp.max(logits_t, axis=0, keepdims=True)
    i1c = jnp.min(jnp.where(logits_t == l1c, et_iota, E), axis=0, keepdims=True)
    masked_c = jnp.where(et_iota == i1c, -jnp.inf, logits_t)
    l2c = jnp.max(masked_c, axis=0, keepdims=True)
    i2c = jnp.min(jnp.where(masked_c == l2c, et_iota, E), axis=0, keepdims=True)
    routed_t = (et_iota == i1c) | (et_iota == i2c)
    active_col = jnp.sum(routed_t.astype(jnp.int32),
                         axis=1, keepdims=True) > 0

    e_row = jax.lax.broadcasted_iota(jnp.int32, (1, E), 1)
    e_col = jax.lax.broadcasted_iota(jnp.int32, (E, 1), 0)
    key_row = jnp.where(active_row, e_row, e_row + E)
    key_col = jnp.where(active_col, e_col, e_col + E)
    rank_col = jnp.sum((key_col > key_row).astype(jnp.int32),
                       axis=1, keepdims=True)
    hit = (rank_col == e_row).astype(jnp.int32)
    ids_vmem[...] = jnp.sum(hit * e_col, axis=0, keepdims=True)
    n_vmem[...] = jnp.sum(active_col.astype(jnp.int32), axis=0,
                          keepdims=True)

    pltpu.make_async_copy(ids_vmem, ids_smem, sem_meta).start()
    pltpu.make_async_copy(n_vmem, n_smem, sem_meta).start()
    out_ref[...] = jnp.zeros_like(out_ref)
    pltpu.make_async_copy(ids_vmem, ids_smem, sem_meta).wait()
    pltpu.make_async_copy(n_vmem, n_smem, sem_meta).wait()
    n = n_smem[0, 0]

    def start_copies(slot, buf):
        eid = ids_smem[0, slot]
        pltpu.make_async_copy(wg_hbm.at[eid], wg_b.at[buf], sems.at[buf, 0]).start()
        pltpu.make_async_copy(wu_hbm.at[eid], wu_b.at[buf], sems.at[buf, 1]).start()
        pltpu.make_async_copy(wd_hbm.at[eid], wd_b.at[buf], sems.at[buf, 2]).start()

    for k in range(K_BUF):

        @pl.when(k < n)
        def _prime(k=k):
            start_copies(k, k)

    def body(i, carry):
        buf = lax.rem(i, K_BUF)
        pltpu.make_async_copy(wg_hbm.at[0], wg_b.at[buf], sems.at[buf, 0]).wait()
        g = jax.lax.dot_general(
            x, wg_b[buf], (((1,), (1,)), ((), ())),
            preferred_element_type=jnp.float32)
        pltpu.make_async_copy(wu_hbm.at[0], wu_b.at[buf], sems.at[buf, 1]).wait()
        u = jax.lax.dot_general(
            x, wu_b[buf], (((1,), (1,)), ((), ())),
            preferred_element_type=jnp.float32)
        h = (g * jax.nn.sigmoid(g)) * u
        pltpu.make_async_copy(wd_hbm.at[0], wd_b.at[buf], sems.at[buf, 2]).wait()
        o = jax.lax.dot_general(
            h, wd_b[buf], (((1,), (1,)), ((), ())),
            preferred_element_type=jnp.float32)
        w_col = jnp.sum(jnp.where(e_iota == ids_smem[0, i], w_scr[...], 0.0),
                        axis=-1, keepdims=True)
        out_ref[...] += o * w_col

        @pl.when(i + K_BUF < n)
        def _next():
            start_copies(i + K_BUF, buf)

        return carry

    lax.fori_loop(0, n, body, 0)


def kernel(hidden_states, gate_w, Wg, Wu, Wd):
    B, S, D = hidden_states.shape
    T = B * S
    E = NUM_EXPERTS
    F = FFN
    x = hidden_states.reshape(T, D)

    out = pl.pallas_call(
        _moe_body,
        in_specs=[
            pl.BlockSpec((T, D), lambda: (0, 0)),
            pl.BlockSpec((E, D), lambda: (0, 0)),
            pl.BlockSpec(memory_space=pl.ANY),
            pl.BlockSpec(memory_space=pl.ANY),
            pl.BlockSpec(memory_space=pl.ANY),
        ],
        out_specs=pl.BlockSpec((T, D), lambda: (0, 0)),
        scratch_shapes=[
            pltpu.VMEM((1, E), jnp.int32),
            pltpu.VMEM((1, 1), jnp.int32),
            pltpu.SMEM((1, E), jnp.int32),
            pltpu.SMEM((1, 1), jnp.int32),
            pltpu.VMEM((T, E), jnp.float32),
            pltpu.VMEM((K_BUF, F, D), jnp.float32),
            pltpu.VMEM((K_BUF, F, D), jnp.float32),
            pltpu.VMEM((K_BUF, D, F), jnp.float32),
            pltpu.SemaphoreType.DMA((K_BUF, 3)),
            pltpu.SemaphoreType.DMA,
        ],
        out_shape=jax.ShapeDtypeStruct((T, D), jnp.float32),
    )(x, gate_w, Wg, Wu, Wd)

    return out.reshape(B, S, D)
